# baseline (speedup 1.0000x reference)
import functools

import jax
import jax.numpy as jnp
from jax import lax
from jax.experimental import pallas as pl
from jax.experimental.pallas import tpu as pltpu

N_NODES = 10000
HID = 256
ROWS = 400
GRID = N_NODES // ROWS


def _proj_relu_body(x_ref, w_ref, b_ref, o_ref):
    acc = jnp.dot(x_ref[...], w_ref[...], preferred_element_type=jnp.float32)
    o_ref[...] = jnp.maximum(acc + b_ref[...], 0.0)


def _proj_relu(x, w, b):
    n, k = x.shape
    m = w.shape[1]
    return pl.pallas_call(
        _proj_relu_body,
        grid=(GRID,),
        in_specs=[
            pl.BlockSpec((ROWS, k), lambda i: (i, 0)),
            pl.BlockSpec((k, m), lambda i: (0, 0)),
            pl.BlockSpec((1, m), lambda i: (0, 0)),
        ],
        out_specs=pl.BlockSpec((ROWS, m), lambda i: (i, 0)),
        out_shape=jax.ShapeDtypeStruct((n, m), jnp.float32),
    )(x, w, b.reshape(1, m))


def _mm_body(x_ref, w_ref, o_ref):
    o_ref[...] = jnp.dot(x_ref[...], w_ref[...],
                         preferred_element_type=jnp.float32)


def _matmul(x, w):
    n, k = x.shape
    m = w.shape[1]
    return pl.pallas_call(
        _mm_body,
        grid=(GRID,),
        in_specs=[
            pl.BlockSpec((ROWS, k), lambda i: (i, 0)),
            pl.BlockSpec((k, m), lambda i: (0, 0)),
        ],
        out_specs=pl.BlockSpec((ROWS, m), lambda i: (i, 0)),
        out_shape=jax.ShapeDtypeStruct((n, m), jnp.float32),
    )(x, w)


def _ln_body(m_ref, r_ref, b_ref, g_ref, bt_ref, o_ref, *, inv_h):
    x = m_ref[...] * inv_h + b_ref[...] + r_ref[...]
    mu = jnp.mean(x, axis=1, keepdims=True)
    var = jnp.mean((x - mu) * (x - mu), axis=1, keepdims=True)
    y = (x - mu) * lax.rsqrt(var + 1e-5) * g_ref[...] + bt_ref[...]
    o_ref[...] = jnp.maximum(y, 0.0)


def _post_gat(msg, res, bias, gamma, beta, n_heads):
    n, c = msg.shape
    return pl.pallas_call(
        functools.partial(_ln_body, inv_h=1.0 / n_heads),
        grid=(GRID,),
        in_specs=[
            pl.BlockSpec((ROWS, c), lambda i: (i, 0)),
            pl.BlockSpec((ROWS, c), lambda i: (i, 0)),
            pl.BlockSpec((1, c), lambda i: (0, 0)),
            pl.BlockSpec((1, c), lambda i: (0, 0)),
            pl.BlockSpec((1, c), lambda i: (0, 0)),
        ],
        out_specs=pl.BlockSpec((ROWS, c), lambda i: (i, 0)),
        out_shape=jax.ShapeDtypeStruct((n, c), jnp.float32),
    )(msg, res, bias.reshape(1, c), gamma.reshape(1, c), beta.reshape(1, c))


PAD_L = 7
KW = 15


def _head_body(xp_ref, oh_ref, wc_ref, cb_ref,
               l1w_ref, l1b_ref, l2w_ref, l2b_ref,
               f1w_ref, f1b_ref, f2w_ref, f2b_ref,
               out_ref, gf_ref, cs_ref):
    i = pl.program_id(0)

    @pl.when(i == 0)
    def _init():
        gf_ref[...] = jnp.zeros_like(gf_ref)
        cs_ref[...] = jnp.zeros_like(cs_ref)

    win = xp_ref[pl.ds(i * ROWS, ROWS + 16), :]
    acc = jnp.zeros((ROWS, HID // 2), jnp.float32)
    for t in range(KW):
        acc = acc + jnp.dot(win[t:t + ROWS, :], wc_ref[t],
                            preferred_element_type=jnp.float32)
    acc = acc + cb_ref[...]
    xc = jnp.maximum(
        jnp.dot(acc, l1w_ref[...], preferred_element_type=jnp.float32)
        + l1b_ref[...], 0.0)
    xc = jnp.dot(xc, l2w_ref[...], preferred_element_type=jnp.float32) \
        + l2b_ref[...]
    cs_ref[...] += jnp.sum(xc, axis=0, keepdims=True)

    xin = win[PAD_L:PAD_L + ROWS, :]
    oh = oh_ref[...]
    gf_ref[...] += lax.dot_general(oh, xin, (((0,), (0,)), ((), ())),
                                   preferred_element_type=jnp.float32)

    @pl.when(i == pl.num_programs(0) - 1)
    def _final():
        gf = gf_ref[...]
        cnn_feat = cs_ref[...] * (1.0 / N_NODES)
        fusion = (gf + cnn_feat) * 0.5
        h = jnp.maximum(
            jnp.dot(fusion, f1w_ref[...], preferred_element_type=jnp.float32)
            + f1b_ref[...], 0.0)
        out_ref[...] = jnp.dot(h, f2w_ref[...],
                               preferred_element_type=jnp.float32) + f2b_ref[...]


def _cnn_head(x, onehot, wc, cb, l1w, l1b, l2w, l2b, f1w, f1b, f2w, f2b):
    xp = jnp.pad(x, ((PAD_L, 16 - PAD_L), (0, 0)))
    c2 = HID // 2
    return pl.pallas_call(
        _head_body,
        grid=(GRID,),
        in_specs=[
            pl.BlockSpec((N_NODES + 16, HID), lambda i: (0, 0)),
            pl.BlockSpec((ROWS, 8), lambda i: (i, 0)),
            pl.BlockSpec((KW, HID, c2), lambda i: (0, 0, 0)),
            pl.BlockSpec((1, c2), lambda i: (0, 0)),
            pl.BlockSpec((c2, 512), lambda i: (0, 0)),
            pl.BlockSpec((1, 512), lambda i: (0, 0)),
            pl.BlockSpec((512, HID), lambda i: (0, 0)),
            pl.BlockSpec((1, HID), lambda i: (0, 0)),
            pl.BlockSpec((HID, HID), lambda i: (0, 0)),
            pl.BlockSpec((1, HID), lambda i: (0, 0)),
            pl.BlockSpec((HID, HID), lambda i: (0, 0)),
            pl.BlockSpec((1, HID), lambda i: (0, 0)),
        ],
        out_specs=pl.BlockSpec((8, HID), lambda i: (0, 0)),
        out_shape=jax.ShapeDtypeStruct((8, HID), jnp.float32),
        scratch_shapes=[
            pltpu.VMEM((8, HID), jnp.float32),
            pltpu.VMEM((1, HID), jnp.float32),
        ],
    )(xp, onehot, wc, cb.reshape(1, c2),
      l1w, l1b.reshape(1, 512), l2w, l2b.reshape(1, HID),
      f1w, f1b.reshape(1, HID), f2w, f2b.reshape(1, HID))


def _edge_phase(h, as_, ad_, s2, d2, n, n_heads):
    e = jax.nn.leaky_relu(as_[s2] + ad_[d2], 0.2)
    emax = jax.ops.segment_max(e, d2, num_segments=n)
    ee = jnp.exp(e - emax[d2])
    den = jax.ops.segment_sum(ee, d2, num_segments=n)
    alpha = ee / (den[d2] + 1e-16)
    out = jnp.zeros((n, h.shape[2]), jnp.float32)
    for hh in range(n_heads):
        out = out + jax.ops.segment_sum(
            h[s2, hh, :] * alpha[:, hh:hh + 1], d2, num_segments=n)
    return out


def _gat_layer(x, s2, d2, W, a_s, a_d, b, gamma, beta, n_heads):
    n = x.shape[0]
    C = W.shape[1] // n_heads
    Wr = W.reshape(W.shape[0], n_heads, C)
    Was = jnp.einsum('khc,hc->kh', Wr, a_s)
    Wad = jnp.einsum('khc,hc->kh', Wr, a_d)
    width = W.shape[1] + 2 * n_heads
    pad_w = (-width) % 128
    Waug = jnp.concatenate(
        [W, Was, Wad, jnp.zeros((W.shape[0], pad_w), jnp.float32)], axis=1)
    out = _matmul(x, Waug)
    h = out[:, :W.shape[1]].reshape(n, n_heads, C)
    as_ = out[:, W.shape[1]:W.shape[1] + n_heads]
    ad_ = out[:, W.shape[1] + n_heads:W.shape[1] + 2 * n_heads]
    msg = _edge_phase(h, as_, ad_, s2, d2, n, n_heads)
    return _post_gat(msg, x, b, gamma, beta, n_heads)


def kernel(emb, edge_index, batch, params):
    p = params
    n = emb.shape[0]
    loop = jnp.arange(n, dtype=edge_index.dtype)
    s2 = jnp.concatenate([edge_index[0], loop])
    d2 = jnp.concatenate([edge_index[1], loop])

    x = _proj_relu(emb, p['W_emb'], p['b_emb'])
    x = _gat_layer(x, s2, d2, p['gat1_W'], p['gat1_as'], p['gat1_ad'],
                   p['gat1_b'], p['ln1_g'], p['ln1_b'], 4)
    x = _gat_layer(x, s2, d2, p['gat2_W'], p['gat2_as'], p['gat2_ad'],
                   p['gat2_b'], p['ln2_g'], p['ln2_b'], 4)
    x = _gat_layer(x, s2, d2, p['gat3_W'], p['gat3_as'], p['gat3_ad'],
                   p['gat3_b'], p['ln3_g'], p['ln3_b'], 1)

    c2 = HID // 2
    wc = jnp.zeros((KW, HID, c2), jnp.float32)
    for w_name, k in (('cw1', 7), ('cw2', 11), ('cw3', 15)):
        w = params[w_name]
        off = (KW - k) // 2
        wt = jnp.transpose(w, (2, 1, 0))
        wc = wc.at[off:off + k].add(wt)
    wc = wc / 3.0
    cb = (p['cb1'] + p['cb2'] + p['cb3']) / 3.0

    onehot = (batch[:, None] == jnp.arange(8, dtype=batch.dtype)[None, :]
              ).astype(jnp.float32)
    counts = jnp.sum(onehot, axis=0, keepdims=True)
    onehot = onehot / jnp.maximum(counts, 1.0)
    enhanced = _cnn_head(x, onehot, wc, cb,
                         p['l1_W'], p['l1_b'], p['l2_W'], p['l2_b'],
                         p['f1_W'], p['f1_b'], p['f2_W'], p['f2_b'])
    return (enhanced, x)
